# Initial kernel scaffold; baseline (speedup 1.0000x reference)
import jax
import jax.numpy as jnp
from jax.experimental import pallas as pl


def kernel(tokens, embedding_weight):
    raise NotImplementedError("write your pallas kernel here")


# speedup vs baseline: 6.3657x; 6.3657x over previous
import functools

import jax
import jax.numpy as jnp
from jax import lax
from jax.experimental import pallas as pl
from jax.experimental.pallas import tpu as pltpu
from jax.experimental.pallas import tpu_sc as plsc

VOCAB = 100000
EMBED_DIM = 128
BATCH = 4096
HIST_LEN = 200

NC = 2
NS = 16
NW = NC * NS

ROWS = BATCH * HIST_LEN
RPW = ROWS // NW
CHUNK = 128
NCHUNK = RPW // CHUNK

_mesh = plsc.VectorSubcoreMesh(core_axis_name="c", subcore_axis_name="s")


@functools.partial(
    pl.kernel,
    out_type=jax.ShapeDtypeStruct((ROWS, EMBED_DIM), jnp.float32),
    mesh=_mesh,
    scratch_types=[
        pltpu.VMEM((NCHUNK, CHUNK), jnp.int32),
        pltpu.VMEM((CHUNK, EMBED_DIM), jnp.float32),
        pltpu.SemaphoreType.DMA,
    ],
)
def _embed_lookup(tok_hbm, table_hbm, out_hbm, idx_v, rows_v, sem):
    wid = lax.axis_index("s") * NC + lax.axis_index("c")
    pltpu.sync_copy(tok_hbm.at[pl.ds(wid * NCHUNK, NCHUNK)], idx_v)
    out_base = wid * RPW

    def step(j, carry):
        pltpu.async_copy(table_hbm.at[idx_v.at[j]], rows_v, sem).wait()
        pltpu.sync_copy(rows_v, out_hbm.at[pl.ds(out_base + j * CHUNK, CHUNK)])
        return carry

    lax.fori_loop(0, NCHUNK, step, 0)


def kernel(tokens, embedding_weight):
    tok = tokens.astype(jnp.int32).reshape(ROWS // CHUNK, CHUNK)
    out = _embed_lookup(tok, embedding_weight)
    return out.reshape(BATCH, HIST_LEN, EMBED_DIM)


# speedup vs baseline: 9.2343x; 1.4506x over previous
import functools

import jax
import jax.numpy as jnp
from jax import lax
from jax.experimental import pallas as pl
from jax.experimental.pallas import tpu as pltpu
from jax.experimental.pallas import tpu_sc as plsc

VOCAB = 100000
EMBED_DIM = 128
BATCH = 4096
HIST_LEN = 200

NC = 2
NS = 16
NW = NC * NS

ROWS = BATCH * HIST_LEN
RPW = ROWS // NW
CHUNK = 128
NCHUNK = RPW // CHUNK
NBUF = 4

_mesh = plsc.VectorSubcoreMesh(core_axis_name="c", subcore_axis_name="s")


@functools.partial(
    pl.kernel,
    out_type=jax.ShapeDtypeStruct((ROWS, EMBED_DIM), jnp.float32),
    mesh=_mesh,
    scratch_types=(
        [pltpu.VMEM((NCHUNK, CHUNK), jnp.int32)]
        + [pltpu.VMEM((CHUNK, EMBED_DIM), jnp.float32) for _ in range(NBUF)]
        + [pltpu.SemaphoreType.DMA for _ in range(2 * NBUF)]
    ),
)
def _embed_lookup(tok_hbm, table_hbm, out_hbm, idx_v, *bufs_and_sems):
    rows = bufs_and_sems[:NBUF]
    gsem = bufs_and_sems[NBUF:2 * NBUF]
    ssem = bufs_and_sems[2 * NBUF:]
    wid = lax.axis_index("s") * NC + lax.axis_index("c")
    pltpu.sync_copy(tok_hbm.at[pl.ds(wid * NCHUNK, NCHUNK)], idx_v)
    out_base = wid * RPW

    def gather_start(j, b):
        pltpu.async_copy(table_hbm.at[idx_v.at[j]], rows[b], gsem[b])

    def gather_wait(b):
        pltpu.make_async_copy(table_hbm.at[idx_v.at[0]], rows[b], gsem[b]).wait()

    def store_start(j, b):
        pltpu.async_copy(rows[b], out_hbm.at[pl.ds(out_base + j * CHUNK, CHUNK)],
                         ssem[b])

    def store_wait(b):
        pltpu.make_async_copy(rows[b], out_hbm.at[pl.ds(out_base, CHUNK)],
                              ssem[b]).wait()

    for b in range(NBUF):
        gather_start(b, b)

    def outer(it, carry):
        j0 = it * NBUF
        for b in range(NBUF):
            gather_wait(b)
            store_start(j0 + b, b)
            store_wait(b)
            gather_start(j0 + b + NBUF, b)
        return carry

    lax.fori_loop(0, (NCHUNK - NBUF) // NBUF, outer, 0)

    for b in range(NBUF):
        j = NCHUNK - NBUF + b
        gather_wait(b)
        store_start(j, b)
        store_wait(b)


def kernel(tokens, embedding_weight):
    tok = tokens.astype(jnp.int32).reshape(ROWS // CHUNK, CHUNK)
    out = _embed_lookup(tok, embedding_weight)
    return out.reshape(BATCH, HIST_LEN, EMBED_DIM)


# speedup vs baseline: 9.2573x; 1.0025x over previous
import functools

import jax
import jax.numpy as jnp
from jax import lax
from jax.experimental import pallas as pl
from jax.experimental.pallas import tpu as pltpu
from jax.experimental.pallas import tpu_sc as plsc

VOCAB = 100000
EMBED_DIM = 128
BATCH = 4096
HIST_LEN = 200

NC = 2
NS = 16
NW = NC * NS

ROWS = BATCH * HIST_LEN
RPW = ROWS // NW
CHUNK = 128
NCHUNK = RPW // CHUNK
NBUF = 5
LOOKAHEAD = 2

_mesh = plsc.VectorSubcoreMesh(core_axis_name="c", subcore_axis_name="s")


@functools.partial(
    pl.kernel,
    out_type=jax.ShapeDtypeStruct((ROWS, EMBED_DIM), jnp.float32),
    mesh=_mesh,
    scratch_types=(
        [pltpu.VMEM((NCHUNK, CHUNK), jnp.int32)]
        + [pltpu.VMEM((CHUNK, EMBED_DIM), jnp.float32) for _ in range(NBUF)]
        + [pltpu.SemaphoreType.DMA for _ in range(2 * NBUF)]
    ),
)
def _embed_lookup(tok_hbm, table_hbm, out_hbm, idx_v, *bufs_and_sems):
    rows = bufs_and_sems[:NBUF]
    gsem = bufs_and_sems[NBUF:2 * NBUF]
    ssem = bufs_and_sems[2 * NBUF:]
    wid = lax.axis_index("s") * NC + lax.axis_index("c")
    pltpu.sync_copy(tok_hbm.at[pl.ds(wid * NCHUNK, NCHUNK)], idx_v)
    out_base = wid * RPW

    def gather_start(j, b):
        pltpu.async_copy(table_hbm.at[idx_v.at[j]], rows[b], gsem[b])

    def gather_wait(b):
        pltpu.make_async_copy(table_hbm.at[idx_v.at[0]], rows[b], gsem[b]).wait()

    def store_start(j, b):
        pltpu.async_copy(rows[b], out_hbm.at[pl.ds(out_base + j * CHUNK, CHUNK)],
                         ssem[b])

    def store_wait(b):
        pltpu.make_async_copy(rows[b], out_hbm.at[pl.ds(out_base, CHUNK)],
                              ssem[b]).wait()

    G = LOOKAHEAD

    for v in range(NBUF):
        gather_start(v, v % NBUF)
        if v >= G:
            gather_wait((v - G) % NBUF)
            store_start(v - G, (v - G) % NBUF)

    def outer(it, carry):
        j0 = NBUF + it * NBUF
        for b in range(NBUF):
            v = j0 + b
            store_wait(b)
            gather_start(v, b)
            gather_wait((b - G) % NBUF)
            store_start(v - G, (b - G) % NBUF)
        return carry

    lax.fori_loop(0, NCHUNK // NBUF - 1, outer, 0)

    for v in range(NCHUNK, NCHUNK + G):
        gather_wait((v - G) % NBUF)
        store_start(v - G, (v - G) % NBUF)
    for v in range(NCHUNK - NBUF, NCHUNK):
        store_wait(v % NBUF)


def kernel(tokens, embedding_weight):
    tok = tokens.astype(jnp.int32).reshape(ROWS // CHUNK, CHUNK)
    out = _embed_lookup(tok, embedding_weight)
    return out.reshape(BATCH, HIST_LEN, EMBED_DIM)
